# baseline (speedup 1.0000x reference)
import functools

import jax
import jax.numpy as jnp
from jax import lax
from jax.experimental import pallas as pl
from jax.experimental.pallas import tpu as pltpu
from jax.experimental.pallas import tpu_sc as plsc

_LAMBDA = 5.0
_Q = 16

_NC, _NS, _L = 2, 16, 16
_NW = _NC * _NS

_SIGN_INT = -(2 ** 31)
_MANT_INT = 0x7FFFFFFF

_R_SC = 32


def _sc_level(cur_ref, out_ref, first, bitpos, lo, hi, obits, r, k, cap):
    lane = lax.broadcasted_iota(jnp.int32, (_L,), 0)
    lane1 = lane + 1
    cand_o = obits | (jnp.int32(1) << bitpos)
    cand_s = cand_o ^ jnp.int32(_SIGN_INT)
    start = lo & ~jnp.int32(15)
    unroll = 4
    trip = (hi - start + jnp.int32(unroll * 16 - 1)) // jnp.int32(
        unroll * 16)

    def inner(j, c):
        aoff, boff = c
        base0 = start + j * (unroll * 16)
        for u in range(unroll):
            base = base0 + u * 16
            kk = cur_ref[pl.ds(base, 16)]
            if first:
                kk = kk ^ ((kk >> 31) & jnp.int32(_MANT_INT))
            pv = jnp.clip(jnp.minimum(lane1, hi - base)
                          - jnp.maximum(lo - base, 0), 0, _L)
            valid = (base + lane >= lo) & (base + lane < hi)
            above = kk >= cand_s
            m_ab = above & valid
            m_be = jnp.logical_not(above) & valid
            pa = plsc.cumsum(m_ab.astype(jnp.int32))
            plsc.store_scatter(out_ref, [aoff + (pa - 1)], kk, mask=m_ab)
            pb = pv - pa
            plsc.store_scatter(out_ref,
                               [jnp.int32(cap - 1) - boff - (pb - 1)], kk,
                               mask=m_be)
            aoff = aoff + plsc.all_reduce_population_count(m_ab)
            boff = boff + plsc.all_reduce_population_count(m_be)
        return aoff, boff

    zero = jnp.zeros((_L,), jnp.int32)
    aoff, boff = lax.fori_loop(0, trip, inner, (zero, zero))
    ca = jnp.max(aoff)
    cb = jnp.max(boff)
    keep_above = (r + ca) >= k
    obits = jnp.where(keep_above, cand_o, obits)
    r = jnp.where(keep_above, r, r + ca)
    lo = jnp.where(keep_above, jnp.int32(0), jnp.int32(cap) - cb)
    hi = jnp.where(keep_above, ca, jnp.int32(cap))
    return lo, hi, obits, r


def _sc_quickselect_kth_key(raw, bufa, bufb, k, cap):
    st = _sc_level(raw, bufa, True, 31, jnp.int32(0), jnp.int32(cap),
                   jnp.int32(0), jnp.int32(0), k, cap)

    def pair(i, st):
        bit = 30 - 2 * i
        lo, hi, obits, r = st
        lo, hi, obits, r = _sc_level(bufa, bufb, False, bit, lo, hi,
                                     obits, r, k, cap)
        return _sc_level(bufb, bufa, False, bit - 1, lo, hi, obits, r,
                         k, cap)

    st = lax.fori_loop(0, 15, pair, st)
    lo, hi, obits, r = _sc_level(bufa, bufb, False, 0, *st, k, cap)
    return obits ^ jnp.int32(_SIGN_INT)


def _sc_topk_keys(hl_i, hlc_i, k, r_sc):
    B, T = hl_i.shape
    rows_per_w = r_sc // _NW
    mesh = plsc.VectorSubcoreMesh(core_axis_name="c", subcore_axis_name="s",
                                  num_cores=_NC, num_subcores=_NS)

    @functools.partial(
        pl.kernel,
        out_type=jax.ShapeDtypeStruct((2, _NW, _L), jnp.int32),
        mesh=mesh,
        scratch_types=[
            pltpu.VMEM((T,), jnp.int32),
            pltpu.VMEM((T + 64,), jnp.int32),
            pltpu.VMEM((T + 64,), jnp.int32),
            pltpu.VMEM((_L,), jnp.int32),
        ],
        compiler_params=pltpu.CompilerParams(needs_layout_passes=False),
    )
    def body(hl_ref, hlc_ref, out_ref, raw, bufa, bufb, res):
        wid = lax.axis_index("c") * _NS + lax.axis_index("s")
        lane = lax.broadcasted_iota(jnp.int32, (_L,), 0)
        for ai, src in enumerate((hl_ref, hlc_ref)):
            def row_body(j, resvec):
                row = wid * rows_per_w + j
                pltpu.sync_copy(src.at[row], raw)
                tkey = _sc_quickselect_kth_key(raw, bufa, bufb, k, T)
                return jnp.where(lane == j, tkey, resvec)

            res[...] = lax.fori_loop(0, rows_per_w, row_body,
                                     jnp.zeros((_L,), jnp.int32))
            pltpu.sync_copy(res, out_ref.at[ai, wid])

    return body(hl_i, hlc_i)


def _monotone_key(x):
    i = lax.bitcast_convert_type(x, jnp.int32)
    return i ^ ((i >> 31) & jnp.int32(_MANT_INT))


def _sct_from_key(skey, x, obits, k):
    t_s = obits ^ jnp.int32(_SIGN_INT)
    gt = skey > t_s
    cnt_gt = jnp.sum(gt.astype(jnp.float32), axis=1, keepdims=True)
    sum_gt = jnp.sum(jnp.where(gt, x, 0.0), axis=1, keepdims=True)
    tbits = t_s ^ ((t_s >> 31) & jnp.int32(_MANT_INT))
    tval = lax.bitcast_convert_type(tbits, jnp.float32)
    del k
    return sum_gt, cnt_gt, tval


def _tc_main_body(k, hl_ref, hlc_ref, dp_ref, shl_ref, chl_ref, thl_ref,
                  shlc_ref, chlc_ref, thlc_ref):
    hl = hl_ref[...]
    hlc = hlc_ref[...]
    s_hl = jax.nn.sigmoid(hl)
    log_sig_hlc = jnp.minimum(hlc, 0.0) - jnp.log1p(jnp.exp(-jnp.abs(hlc)))
    dp_ref[...] = jnp.sum(s_hl * log_sig_hlc, axis=1, keepdims=True)
    ka = _monotone_key(hl)
    kb = _monotone_key(hlc)
    R = hl.shape[0]
    sign = jnp.int32(_SIGN_INT)

    def body(i, carry):
        oa, ob = carry
        bit = jnp.int32(1) << (31 - i)
        ca = (oa | bit) ^ sign
        cb = (ob | bit) ^ sign
        cnt_a = jnp.sum((ka >= ca).astype(jnp.int32), axis=1, keepdims=True)
        cnt_b = jnp.sum((kb >= cb).astype(jnp.int32), axis=1, keepdims=True)
        return (jnp.where(cnt_a >= k, oa | bit, oa),
                jnp.where(cnt_b >= k, ob | bit, ob))

    zero = jnp.zeros((R, 1), jnp.int32)
    oa, ob = lax.fori_loop(0, 32, body, (zero, zero))
    shl_ref[...], chl_ref[...], thl_ref[...] = _sct_from_key(ka, hl, oa, k)
    shlc_ref[...], chlc_ref[...], thlc_ref[...] = _sct_from_key(
        kb, hlc, ob, k)


def _tc_fixup_body(hl_ref, hlc_ref, thl_ref, thlc_ref,
                   dp_ref, shl_ref, chl_ref, shlc_ref, chlc_ref):
    hl = hl_ref[...]
    hlc = hlc_ref[...]
    s_hl = jax.nn.sigmoid(hl)
    log_sig_hlc = jnp.minimum(hlc, 0.0) - jnp.log1p(jnp.exp(-jnp.abs(hlc)))
    dp_ref[...] = jnp.sum(s_hl * log_sig_hlc, axis=1, keepdims=True)
    for x, t_ref, s_ref, c_ref in ((hl, thl_ref, shl_ref, chl_ref),
                                   (hlc, thlc_ref, shlc_ref, chlc_ref)):
        gt = x > t_ref[...]
        s_ref[...] = jnp.sum(jnp.where(gt, x, 0.0), axis=1, keepdims=True)
        c_ref[...] = jnp.sum(gt.astype(jnp.float32), axis=1, keepdims=True)


def _finish_body(k, b, shl_ref, chl_ref, thl_ref, shlc_ref, chlc_ref,
                 thlc_ref, dp_ref, y_ref,
                 total_ref, distill_ref, mil_hl_ref, mil_hlc_ref):
    y = y_ref[...]

    def bce_mean(s, c, t):
        x = (s + (k - c) * t) * (1.0 / k)
        return jnp.mean(jnp.maximum(x, 0.0) - x * y
                        + jnp.log1p(jnp.exp(-jnp.abs(x))))

    mil_hl = bce_mean(shl_ref[...], chl_ref[...], thl_ref[...])
    mil_hlc = bce_mean(shlc_ref[...], chlc_ref[...], thlc_ref[...])
    distill = -jnp.sum(dp_ref[...]) * (1.0 / b)
    total_ref[0] = _LAMBDA * distill + mil_hlc + mil_hl
    distill_ref[0] = distill
    mil_hl_ref[0] = mil_hl
    mil_hlc_ref[0] = mil_hlc


def kernel(logits_hl, logits_hlc, bag_labels):
    B, T, _ = logits_hl.shape
    k = max(T // _Q, 1)
    hl = logits_hl.reshape(B, T)
    hlc = logits_hlc.reshape(B, T)

    hl_i = lax.bitcast_convert_type(hl, jnp.int32)
    hlc_i = lax.bitcast_convert_type(hlc, jnp.int32)
    keys = _sc_topk_keys(hl_i, hlc_i, k, _R_SC)
    rows_per_w = _R_SC // _NW
    tkeys = keys[:, :, :rows_per_w].reshape(2, _R_SC)
    tvals_sc = lax.bitcast_convert_type(
        tkeys ^ ((tkeys >> 31) & jnp.int32(_MANT_INT)), jnp.float32)

    rows = 32
    r_tc = B - _R_SC
    off = _R_SC // rows
    spec_in = pl.BlockSpec((rows, T), lambda i: (i + off, 0))
    spec_row = pl.BlockSpec((rows, 1), lambda i: (i, 0))
    dp_tc, s1, c1, t1, s2, c2, t2 = pl.pallas_call(
        functools.partial(_tc_main_body, k),
        grid=(r_tc // rows,),
        in_specs=[spec_in, spec_in],
        out_specs=[spec_row] * 7,
        out_shape=[jax.ShapeDtypeStruct((r_tc, 1), jnp.float32)] * 7,
    )(hl, hlc)

    rows_b = 16
    spec_in_b = pl.BlockSpec((rows_b, T), lambda i: (i, 0))
    spec_row_b = pl.BlockSpec((rows_b, 1), lambda i: (i, 0))
    dp_sc, s1s, c1s, s2s, c2s = pl.pallas_call(
        _tc_fixup_body,
        grid=(_R_SC // rows_b,),
        in_specs=[spec_in_b, spec_in_b, spec_row_b, spec_row_b],
        out_specs=[spec_row_b] * 5,
        out_shape=[jax.ShapeDtypeStruct((_R_SC, 1), jnp.float32)] * 5,
    )(hl, hlc, tvals_sc[0].reshape(_R_SC, 1), tvals_sc[1].reshape(_R_SC, 1))

    dp = jnp.concatenate([dp_sc, dp_tc]).reshape(1, B)
    s_hl = jnp.concatenate([s1s, s1]).reshape(1, B)
    c_hl = jnp.concatenate([c1s, c1]).reshape(1, B)
    t_hl = jnp.concatenate([tvals_sc[0].reshape(_R_SC, 1), t1]).reshape(1, B)
    s_hlc = jnp.concatenate([s2s, s2]).reshape(1, B)
    c_hlc = jnp.concatenate([c2s, c2]).reshape(1, B)
    t_hlc = jnp.concatenate([tvals_sc[1].reshape(_R_SC, 1), t2]).reshape(1, B)

    y = bag_labels.astype(jnp.float32).reshape(1, B)
    total, distill, mil_hl, mil_hlc = pl.pallas_call(
        functools.partial(_finish_body, k, B),
        in_specs=[pl.BlockSpec((1, B), lambda: (0, 0))] * 8,
        out_specs=[pl.BlockSpec(memory_space=pltpu.SMEM)] * 4,
        out_shape=[jax.ShapeDtypeStruct((1,), jnp.float32)] * 4,
    )(s_hl, c_hl, t_hl, s_hlc, c_hlc, t_hlc, dp, y)

    return (total.reshape(()), distill.reshape(()),
            mil_hl.reshape(()), mil_hlc.reshape(()))


# speedup vs baseline: 1.6298x; 1.6298x over previous
import functools

import jax
import jax.numpy as jnp
from jax.experimental import pallas as pl
from jax.experimental.pallas import tpu as pltpu

_LAMBDA = 5.0
_Q = 16


def _monotone_key(x):
    i = jax.lax.bitcast_convert_type(x, jnp.int32)
    return i ^ ((i >> 31) & jnp.int32(0x7FFFFFFF))


def _topk_sum_rows2(xa, xb, k):
    ka = _monotone_key(xa)
    kb = _monotone_key(xb)
    R = xa.shape[0]
    sign = jnp.int32(-(2**31))

    def body(i, carry):
        oa, ob = carry
        bit = jnp.int32(1) << (31 - i)
        ca = (oa | bit) ^ sign
        cb = (ob | bit) ^ sign
        cnt_a = jnp.sum((ka >= ca).astype(jnp.int32), axis=1, keepdims=True)
        cnt_b = jnp.sum((kb >= cb).astype(jnp.int32), axis=1, keepdims=True)
        return (jnp.where(cnt_a >= k, oa | bit, oa),
                jnp.where(cnt_b >= k, ob | bit, ob))

    zero = jnp.zeros((R, 1), jnp.int32)
    oa, ob = jax.lax.fori_loop(0, 32, body, (zero, zero))

    def finish(skey, obits):
        t_s = obits ^ sign
        gt = skey > t_s
        cnt_gt = jnp.sum(gt.astype(jnp.int32), axis=1, keepdims=True)
        x = jax.lax.bitcast_convert_type(
            skey ^ ((skey >> 31) & jnp.int32(0x7FFFFFFF)), jnp.float32)
        sum_gt = jnp.sum(jnp.where(gt, x, 0.0), axis=1, keepdims=True)
        tbits = t_s ^ ((t_s >> 31) & jnp.int32(0x7FFFFFFF))
        tval = jax.lax.bitcast_convert_type(tbits, jnp.float32)
        return sum_gt + (k - cnt_gt).astype(jnp.float32) * tval

    return finish(ka, oa), finish(kb, ob)


def _main_body(k, hl_ref, hlc_ref, tks_hl_ref, tks_hlc_ref, dp_ref):
    hl = hl_ref[...]
    hlc = hlc_ref[...]
    s_hl = jax.nn.sigmoid(hl)
    log_sig_hlc = jnp.minimum(hlc, 0.0) - jnp.log1p(jnp.exp(-jnp.abs(hlc)))
    dp_ref[...] = jnp.sum(s_hl * log_sig_hlc, axis=1, keepdims=True)
    tks_hl, tks_hlc = _topk_sum_rows2(hl, hlc, k)
    tks_hl_ref[...] = tks_hl
    tks_hlc_ref[...] = tks_hlc


def _finish_body(k, b, tks_hl_ref, tks_hlc_ref, dp_ref, y_ref,
                 total_ref, distill_ref, mil_hl_ref, mil_hlc_ref):
    y = y_ref[...]

    def bce_mean(x):
        return jnp.mean(jnp.maximum(x, 0.0) - x * y
                        + jnp.log1p(jnp.exp(-jnp.abs(x))))

    mil_hl = bce_mean(tks_hl_ref[...] * (1.0 / k))
    mil_hlc = bce_mean(tks_hlc_ref[...] * (1.0 / k))
    distill = -jnp.sum(dp_ref[...]) * (1.0 / b)
    total_ref[0] = _LAMBDA * distill + mil_hlc + mil_hl
    distill_ref[0] = distill
    mil_hl_ref[0] = mil_hl
    mil_hlc_ref[0] = mil_hlc


def kernel(logits_hl, logits_hlc, bag_labels):
    B, T, _ = logits_hl.shape
    k = max(T // _Q, 1)
    rows = 64
    grid = B // rows
    hl = logits_hl.reshape(B, T)
    hlc = logits_hlc.reshape(B, T)

    tks_hl, tks_hlc, dp = pl.pallas_call(
        functools.partial(_main_body, k),
        grid=(grid,),
        in_specs=[
            pl.BlockSpec((rows, T), lambda i: (i, 0)),
            pl.BlockSpec((rows, T), lambda i: (i, 0)),
        ],
        out_specs=[
            pl.BlockSpec((rows, 1), lambda i: (i, 0)),
            pl.BlockSpec((rows, 1), lambda i: (i, 0)),
            pl.BlockSpec((rows, 1), lambda i: (i, 0)),
        ],
        out_shape=[
            jax.ShapeDtypeStruct((B, 1), jnp.float32),
            jax.ShapeDtypeStruct((B, 1), jnp.float32),
            jax.ShapeDtypeStruct((B, 1), jnp.float32),
        ],
    )(hl, hlc)

    y = bag_labels.astype(jnp.float32).reshape(1, B)
    total, distill, mil_hl, mil_hlc = pl.pallas_call(
        functools.partial(_finish_body, k, B),
        in_specs=[
            pl.BlockSpec((1, B), lambda: (0, 0)),
            pl.BlockSpec((1, B), lambda: (0, 0)),
            pl.BlockSpec((1, B), lambda: (0, 0)),
            pl.BlockSpec((1, B), lambda: (0, 0)),
        ],
        out_specs=[
            pl.BlockSpec(memory_space=pltpu.SMEM),
            pl.BlockSpec(memory_space=pltpu.SMEM),
            pl.BlockSpec(memory_space=pltpu.SMEM),
            pl.BlockSpec(memory_space=pltpu.SMEM),
        ],
        out_shape=[jax.ShapeDtypeStruct((1,), jnp.float32)] * 4,
    )(tks_hl.reshape(1, B), tks_hlc.reshape(1, B), dp.reshape(1, B), y)

    return (total.reshape(()), distill.reshape(()),
            mil_hl.reshape(()), mil_hlc.reshape(()))
